# baseline (speedup 1.0000x reference)
import functools

import jax
import jax.numpy as jnp
from jax import lax
from jax.experimental import pallas as pl
from jax.experimental.pallas import tpu as pltpu
from jax.experimental.pallas import tpu_sc as plsc

NC = 2
NS = 16
NW = NC * NS
LANES = 128


def _mesh():
  return plsc.VectorSubcoreMesh(core_axis_name="c", subcore_axis_name="s")


def _stage_of(slc, cap=512):
  return next(s for s in range(cap, 7, -8) if slc % s == 0)


def _deg_kernel(n_pad, rows_pt, rb):
  slc = n_pad // NS
  stage = _stage_of(slc)
  n_stage = slc // stage

  @functools.partial(
      pl.kernel,
      out_type=jax.ShapeDtypeStruct((NC * n_pad,), jnp.float32),
      mesh=_mesh(),
      compiler_params=pltpu.CompilerParams(use_tc_tiling_on_sc=False),
      scratch_types=[
          pltpu.VMEM_SHARED((n_pad,), jnp.float32),
          pltpu.VMEM((2, rb, LANES), jnp.int32),
          pltpu.VMEM((LANES,), jnp.float32),
          pltpu.VMEM((stage,), jnp.float32),
          pltpu.SemaphoreType.DMA,
      ],
  )
  def k(dst_hbm, out_hbm, deg_sp, idx_v, ones_v, stg1, ssem):
    c = lax.axis_index("c")
    s = lax.axis_index("s")
    wid = c * NS + s
    r0s = s * slc

    @pl.loop(0, stage // 16)
    def _(i):
      stg1[pl.ds(i * 16, 16)] = jnp.zeros((16,), jnp.float32)

    @pl.loop(0, n_stage)
    def _(i):
      pltpu.sync_copy(stg1, deg_sp.at[pl.ds(r0s + i * stage, stage)])

    for j in range(LANES // 16):
      ones_v[pl.ds(j * 16, 16)] = jnp.full((16,), 1.0, jnp.float32)
    plsc.subcore_barrier()
    row_base = wid * rows_pt
    nblk = rows_pt // rb
    row_cap = NW * rows_pt - rb

    def load(b, half):
      off = lax.min(row_base + b * rb, row_cap)
      pltpu.sync_copy(dst_hbm.at[pl.ds(off, rb)], idx_v.at[half])

    def fire(half):
      return [pltpu.async_copy(ones_v, deg_sp.at[idx_v.at[half, r]], ssem,
                               add=True) for r in range(rb)]

    load(0, 0)

    @pl.loop(0, nblk // 2)
    def _(t):
      b0 = 2 * t
      s0 = fire(0)
      load(b0 + 1, 1)
      s1 = fire(1)
      for d in s0:
        d.wait()
      load(b0 + 2, 0)
      for d in s1:
        d.wait()

    if nblk % 2:
      st = fire(0)
      for d in st:
        d.wait()

    plsc.subcore_barrier()

    @pl.loop(0, n_stage)
    def _(i):
      pltpu.sync_copy(deg_sp.at[pl.ds(r0s + i * stage, stage)], stg1)
      pltpu.sync_copy(stg1, out_hbm.at[pl.ds(c * n_pad + r0s + i * stage,
                                             stage)])

  return k


def _main_kernel(n_pad, rows_pt, rb):
  slc = n_pad // NS
  stage = _stage_of(slc)
  n_stage = slc // stage

  @functools.partial(
      pl.kernel,
      out_type=(jax.ShapeDtypeStruct((NC * n_pad, 8), jnp.float32),
                jax.ShapeDtypeStruct((NC * n_pad,), jnp.float32)),
      mesh=_mesh(),
      compiler_params=pltpu.CompilerParams(use_tc_tiling_on_sc=False),
      scratch_types=[
          pltpu.VMEM_SHARED((n_pad, 8), jnp.float32),
          pltpu.VMEM_SHARED((n_pad, 8), jnp.float32),
          pltpu.VMEM_SHARED((n_pad,), jnp.float32),
          pltpu.VMEM_SHARED((n_pad,), jnp.float32),
          pltpu.VMEM((2, rb, LANES), jnp.int32),
          pltpu.VMEM((2, rb, LANES), jnp.int32),
          pltpu.VMEM((1, rb, LANES, 8), jnp.float32),
          pltpu.VMEM((1, rb, LANES), jnp.float32),
          pltpu.VMEM((stage, 8), jnp.float32),
          pltpu.VMEM((stage,), jnp.float32),
          pltpu.SemaphoreType.DMA,
          pltpu.SemaphoreType.DMA,
          pltpu.SemaphoreType.DMA,
          pltpu.SemaphoreType.DMA,
      ],
  )
  def k(src_hbm, dst_hbm, y_hbm, dinv_hbm, z8_hbm,
        s_out, c_out, y_sp, s_sp, c_sp, dinv_sp, sidx, didx, ybufs, cvals,
        stg8, stg1, gsem, dsem, ssem, csem):
    c = lax.axis_index("c")
    s = lax.axis_index("s")
    wid = c * NS + s
    r0s = s * slc

    @pl.loop(0, stage // 16)
    def _(i):
      stg1[pl.ds(i * 16, 16)] = jnp.zeros((16,), jnp.float32)

    pltpu.sync_copy(z8_hbm, stg8)

    @pl.loop(0, n_stage)
    def _(i):
      pltpu.sync_copy(stg8, s_sp.at[pl.ds(r0s + i * stage, stage)])
      pltpu.sync_copy(stg1, c_sp.at[pl.ds(r0s + i * stage, stage)])

    @pl.loop(0, n_stage)
    def _(i):
      pltpu.sync_copy(y_hbm.at[pl.ds(r0s + i * stage, stage)], stg8)
      pltpu.sync_copy(stg8, y_sp.at[pl.ds(r0s + i * stage, stage)])
      pltpu.sync_copy(dinv_hbm.at[pl.ds(r0s + i * stage, stage)], stg1)
      pltpu.sync_copy(stg1, dinv_sp.at[pl.ds(r0s + i * stage, stage)])

    plsc.subcore_barrier()
    row_base = wid * rows_pt
    nblk = rows_pt // rb
    row_cap = NW * rows_pt - rb

    def load(b, half):
      off = lax.min(row_base + b * rb, row_cap)
      pltpu.sync_copy(src_hbm.at[pl.ds(off, rb)], sidx.at[half])
      pltpu.sync_copy(dst_hbm.at[pl.ds(off, rb)], didx.at[half])

    def fire_gathers(h):
      return ([pltpu.async_copy(y_sp.at[sidx.at[h, r]], ybufs.at[0, r], gsem)
               for r in range(rb)]
              + [pltpu.async_copy(dinv_sp.at[didx.at[h, r]], cvals.at[0, r],
                                  dsem) for r in range(rb)])

    def fire_scatters(h):
      return ([pltpu.async_copy(ybufs.at[0, r], s_sp.at[didx.at[h, r]], ssem,
                                add=True) for r in range(rb)]
              + [pltpu.async_copy(cvals.at[0, r], c_sp.at[sidx.at[h, r]],
                                  csem, add=True) for r in range(rb)])

    def step(b0, h):
      g = fire_gathers(h)
      load(b0 + 1, 1 - h)
      for d in g:
        d.wait()
      s = fire_scatters(h)
      for d in s:
        d.wait()

    load(0, 0)

    @pl.loop(0, nblk // 2)
    def _(t):
      step(2 * t, 0)
      step(2 * t + 1, 1)

    plsc.subcore_barrier()

    @pl.loop(0, n_stage)
    def _(i):
      pltpu.sync_copy(s_sp.at[pl.ds(r0s + i * stage, stage)], stg8)
      pltpu.sync_copy(stg8, s_out.at[pl.ds(c * n_pad + r0s + i * stage,
                                           stage)])
      pltpu.sync_copy(c_sp.at[pl.ds(r0s + i * stage, stage)], stg1)
      pltpu.sync_copy(stg1, c_out.at[pl.ds(c * n_pad + r0s + i * stage,
                                           stage)])

  return k


def _norm_tc(n_pad, n, blk):
  nb = n_pad // blk

  def body(p0_ref, p1_ref, x_ref, dinv_ref, y_ref):
    i = pl.program_id(0)
    deg = p0_ref[...] + p1_ref[...] + 1.0
    dinv = lax.rsqrt(deg)
    rid = lax.broadcasted_iota(jnp.int32, (blk,), 0) + i * blk
    dinv = jnp.where(rid < n, dinv, 0.0)
    dinv_ref[...] = dinv
    y_ref[...] = dinv.reshape(blk, 1) * x_ref[...]

  return pl.pallas_call(
      body,
      grid=(nb,),
      in_specs=[
          pl.BlockSpec((blk,), lambda i: (i,)),
          pl.BlockSpec((blk,), lambda i: (i + nb,)),
          pl.BlockSpec((blk, 8), lambda i: (i, 0)),
      ],
      out_specs=[
          pl.BlockSpec((blk,), lambda i: (i,)),
          pl.BlockSpec((blk, 8), lambda i: (i, 0)),
      ],
      out_shape=(jax.ShapeDtypeStruct((n_pad,), jnp.float32),
                 jax.ShapeDtypeStruct((n_pad, 8), jnp.float32)),
  )


def _fuse_tc(n_pad, n, blk):
  nb = n_pad // blk

  def body(s0_ref, s1_ref, x_ref, dv_ref, c0_ref, c1_ref,
           w1_ref, b1_ref, w2_ref, b2_ref, st_ref, ws_ref, bs_ref,
           wc1_ref, wc2_ref, bc_ref, out_ref, acc):
    i = pl.program_id(0)

    @pl.when(i == 0)
    def _():
      acc[...] = jnp.zeros_like(acc)

    dv = dv_ref[...]
    dvc = dv.reshape(blk, 1)
    a1 = dvc * (s0_ref[...] + s1_ref[...] + dvc * x_ref[...])
    h1 = jnp.maximum(
        jnp.dot(a1, w1_ref[...], preferred_element_type=jnp.float32)
        + b1_ref[...], 0.0)
    w = dv * (c0_ref[...] + c1_ref[...] + dv) * (1.0 / n)
    acc[...] += jnp.dot(w.reshape(1, blk), h1,
                        preferred_element_type=jnp.float32)

    @pl.when(i == nb - 1)
    def _():
      pooled = jnp.dot(acc[...], w2_ref[...],
                       preferred_element_type=jnp.float32) + b2_ref[...]
      sf = jnp.maximum(
          jnp.dot(st_ref[...], ws_ref[...],
                  preferred_element_type=jnp.float32) + bs_ref[...], 0.0)
      out_ref[...] = (
          jnp.dot(pooled, wc1_ref[...], preferred_element_type=jnp.float32)
          + jnp.dot(sf, wc2_ref[...], preferred_element_type=jnp.float32)
          + bc_ref[...])

  full = lambda shape: pl.BlockSpec(shape, lambda i: tuple(0 for _ in shape))
  return pl.pallas_call(
      body,
      grid=(nb,),
      in_specs=[
          pl.BlockSpec((blk, 8), lambda i: (i, 0)),
          pl.BlockSpec((blk, 8), lambda i: (i + nb, 0)),
          pl.BlockSpec((blk, 8), lambda i: (i, 0)),
          pl.BlockSpec((blk,), lambda i: (i,)),
          pl.BlockSpec((blk,), lambda i: (i,)),
          pl.BlockSpec((blk,), lambda i: (i + nb,)),
          full((8, 64)), full((1, 64)),
          full((64, 64)), full((1, 64)),
          full((1, 8)), full((8, 64)), full((1, 64)),
          full((64, 2)), full((64, 2)), full((1, 2)),
      ],
      out_specs=pl.BlockSpec((1, 2), lambda i: (0, 0)),
      out_shape=jax.ShapeDtypeStruct((1, 2), jnp.float32),
      scratch_shapes=[pltpu.VMEM((1, 64), jnp.float32)],
  )


def kernel(x, edge_index, state, W1, b1, W2, b2, Ws, bs, Wc, bc):
  n, _ = x.shape
  e = edge_index.shape[1]
  n_pad = ((n + 1 + 1023) // 1024) * 1024
  rows_pt8 = (e + NW * LANES - 1) // (NW * LANES)
  rows_pt = ((rows_pt8 + 7) // 8) * 8
  e_pad = NW * rows_pt * LANES
  rb = 8

  sent = (n + (jnp.arange(e_pad - e, dtype=jnp.int32) % (n_pad - n)))
  src2d = jnp.concatenate([edge_index[0], sent]).reshape(-1, LANES)
  dst2d = jnp.concatenate([edge_index[1], sent]).reshape(-1, LANES)
  x_pad = jnp.pad(x, ((0, n_pad - n), (0, 0)))
  zeros8 = jnp.zeros((_stage_of(n_pad // NS), 8), jnp.float32)

  degp = _deg_kernel(n_pad, rows_pt, 16)(dst2d)

  blk = next(n_pad // nb for nb in (14, 16, 8, 4, 2, 1)
             if n_pad % (nb * 1024) == 0)
  dinv, y = _norm_tc(n_pad, n, blk)(degp, degp, x_pad)

  sp, cp = _main_kernel(n_pad, rows_pt, rb)(src2d, dst2d, y, dinv, zeros8)

  out = _fuse_tc(n_pad, n, blk)(
      sp, sp, x_pad, dinv, cp, cp,
      W1, b1.reshape(1, -1), W2, b2.reshape(1, -1),
      state, Ws, bs.reshape(1, -1), Wc[:64], Wc[64:], bc.reshape(1, -1))
  return out


# speedup vs baseline: 1.0124x; 1.0124x over previous
import functools

import jax
import jax.numpy as jnp
from jax import lax
from jax.experimental import pallas as pl
from jax.experimental.pallas import tpu as pltpu
from jax.experimental.pallas import tpu_sc as plsc

NC = 2
NS = 16
NW = NC * NS
LANES = 128


def _mesh():
  return plsc.VectorSubcoreMesh(core_axis_name="c", subcore_axis_name="s")


def _stage_of(slc, cap=512):
  return next(s for s in range(cap, 7, -8) if slc % s == 0)


def _deg_kernel(n_pad, rows_pt, rb):
  slc = n_pad // NS
  stage = _stage_of(slc)
  n_stage = slc // stage

  @functools.partial(
      pl.kernel,
      out_type=jax.ShapeDtypeStruct((NC * n_pad,), jnp.float32),
      mesh=_mesh(),
      compiler_params=pltpu.CompilerParams(use_tc_tiling_on_sc=False),
      scratch_types=[
          pltpu.VMEM_SHARED((n_pad,), jnp.float32),
          pltpu.VMEM((2, rb, LANES), jnp.int32),
          pltpu.VMEM((LANES,), jnp.float32),
          pltpu.VMEM((stage,), jnp.float32),
          pltpu.SemaphoreType.DMA,
      ],
  )
  def k(dst_hbm, out_hbm, deg_sp, idx_v, ones_v, stg1, ssem):
    c = lax.axis_index("c")
    s = lax.axis_index("s")
    wid = c * NS + s
    r0s = s * slc

    @pl.loop(0, stage // 16)
    def _(i):
      stg1[pl.ds(i * 16, 16)] = jnp.zeros((16,), jnp.float32)

    @pl.loop(0, n_stage)
    def _(i):
      pltpu.sync_copy(stg1, deg_sp.at[pl.ds(r0s + i * stage, stage)])

    for j in range(LANES // 16):
      ones_v[pl.ds(j * 16, 16)] = jnp.full((16,), 1.0, jnp.float32)
    plsc.subcore_barrier()
    row_base = wid * rows_pt
    nblk = rows_pt // rb
    row_cap = NW * rows_pt - rb

    def load(b, half):
      off = lax.min(row_base + b * rb, row_cap)
      pltpu.sync_copy(dst_hbm.at[pl.ds(off, rb)], idx_v.at[half])

    def fire(half):
      return [pltpu.async_copy(ones_v, deg_sp.at[idx_v.at[half, r]], ssem,
                               add=True) for r in range(rb)]

    load(0, 0)

    @pl.loop(0, nblk // 2)
    def _(t):
      b0 = 2 * t
      s0 = fire(0)
      load(b0 + 1, 1)
      s1 = fire(1)
      for d in s0:
        d.wait()
      load(b0 + 2, 0)
      for d in s1:
        d.wait()

    if nblk % 2:
      st = fire(0)
      for d in st:
        d.wait()

    plsc.subcore_barrier()

    @pl.loop(0, n_stage)
    def _(i):
      pltpu.sync_copy(deg_sp.at[pl.ds(r0s + i * stage, stage)], stg1)
      pltpu.sync_copy(stg1, out_hbm.at[pl.ds(c * n_pad + r0s + i * stage,
                                             stage)])

  return k


def _main_kernel(n_pad, rows_pt, rb):
  slc = n_pad // NS
  stage = _stage_of(slc)
  n_stage = slc // stage

  @functools.partial(
      pl.kernel,
      out_type=(jax.ShapeDtypeStruct((NC * n_pad, 8), jnp.float32),
                jax.ShapeDtypeStruct((NC * n_pad,), jnp.float32)),
      mesh=_mesh(),
      compiler_params=pltpu.CompilerParams(use_tc_tiling_on_sc=False),
      scratch_types=[
          pltpu.VMEM_SHARED((n_pad, 8), jnp.float32),
          pltpu.VMEM_SHARED((n_pad, 8), jnp.float32),
          pltpu.VMEM_SHARED((n_pad,), jnp.float32),
          pltpu.VMEM_SHARED((n_pad,), jnp.float32),
          pltpu.VMEM((1, rb, LANES), jnp.int32),
          pltpu.VMEM((1, rb, LANES), jnp.int32),
          pltpu.VMEM((1, rb, LANES, 8), jnp.float32),
          pltpu.VMEM((1, rb, LANES), jnp.float32),
          pltpu.VMEM((stage, 8), jnp.float32),
          pltpu.VMEM((stage,), jnp.float32),
          pltpu.SemaphoreType.DMA,
          pltpu.SemaphoreType.DMA,
          pltpu.SemaphoreType.DMA,
          pltpu.SemaphoreType.DMA,
      ],
  )
  def k(src_hbm, dst_hbm, y_hbm, dinv_hbm, z8_hbm,
        s_out, c_out, y_sp, s_sp, c_sp, dinv_sp, sidx, didx, ybufs, cvals,
        stg8, stg1, gsem, dsem, ssem, csem):
    c = lax.axis_index("c")
    s = lax.axis_index("s")
    wid = c * NS + s
    r0s = s * slc

    @pl.loop(0, stage // 16)
    def _(i):
      stg1[pl.ds(i * 16, 16)] = jnp.zeros((16,), jnp.float32)

    pltpu.sync_copy(z8_hbm, stg8)

    @pl.loop(0, n_stage)
    def _(i):
      pltpu.sync_copy(stg8, s_sp.at[pl.ds(r0s + i * stage, stage)])
      pltpu.sync_copy(stg1, c_sp.at[pl.ds(r0s + i * stage, stage)])

    @pl.loop(0, n_stage)
    def _(i):
      pltpu.sync_copy(y_hbm.at[pl.ds(r0s + i * stage, stage)], stg8)
      pltpu.sync_copy(stg8, y_sp.at[pl.ds(r0s + i * stage, stage)])
      pltpu.sync_copy(dinv_hbm.at[pl.ds(r0s + i * stage, stage)], stg1)
      pltpu.sync_copy(stg1, dinv_sp.at[pl.ds(r0s + i * stage, stage)])

    plsc.subcore_barrier()
    row_base = wid * rows_pt
    nblk = rows_pt // rb
    row_cap = NW * rows_pt - rb

    def load(b, half):
      off = lax.min(row_base + b * rb, row_cap)
      pltpu.sync_copy(src_hbm.at[pl.ds(off, rb)], sidx.at[half])
      pltpu.sync_copy(dst_hbm.at[pl.ds(off, rb)], didx.at[half])

    def fire_gathers(h):
      return ([pltpu.async_copy(y_sp.at[sidx.at[h, r]], ybufs.at[0, r], gsem)
               for r in range(rb)]
              + [pltpu.async_copy(dinv_sp.at[didx.at[h, r]], cvals.at[0, r],
                                  dsem) for r in range(rb)])

    def fire_scatters(h):
      return ([pltpu.async_copy(ybufs.at[0, r], s_sp.at[didx.at[h, r]], ssem,
                                add=True) for r in range(rb)]
              + [pltpu.async_copy(cvals.at[0, r], c_sp.at[sidx.at[h, r]],
                                  csem, add=True) for r in range(rb)])

    @pl.loop(0, nblk)
    def _(b):
      load(b, 0)
      g = fire_gathers(0)
      for d in g:
        d.wait()
      s = fire_scatters(0)
      for d in s:
        d.wait()

    plsc.subcore_barrier()

    @pl.loop(0, n_stage)
    def _(i):
      pltpu.sync_copy(s_sp.at[pl.ds(r0s + i * stage, stage)], stg8)
      pltpu.sync_copy(stg8, s_out.at[pl.ds(c * n_pad + r0s + i * stage,
                                           stage)])
      pltpu.sync_copy(c_sp.at[pl.ds(r0s + i * stage, stage)], stg1)
      pltpu.sync_copy(stg1, c_out.at[pl.ds(c * n_pad + r0s + i * stage,
                                           stage)])

  return k


def _norm_tc(n_pad, n, blk):
  nb = n_pad // blk

  def body(p0_ref, p1_ref, x_ref, dinv_ref, y_ref):
    i = pl.program_id(0)
    deg = p0_ref[...] + p1_ref[...] + 1.0
    dinv = lax.rsqrt(deg)
    rid = lax.broadcasted_iota(jnp.int32, (blk,), 0) + i * blk
    dinv = jnp.where(rid < n, dinv, 0.0)
    dinv_ref[...] = dinv
    y_ref[...] = dinv.reshape(blk, 1) * x_ref[...]

  return pl.pallas_call(
      body,
      grid=(nb,),
      in_specs=[
          pl.BlockSpec((blk,), lambda i: (i,)),
          pl.BlockSpec((blk,), lambda i: (i + nb,)),
          pl.BlockSpec((blk, 8), lambda i: (i, 0)),
      ],
      out_specs=[
          pl.BlockSpec((blk,), lambda i: (i,)),
          pl.BlockSpec((blk, 8), lambda i: (i, 0)),
      ],
      out_shape=(jax.ShapeDtypeStruct((n_pad,), jnp.float32),
                 jax.ShapeDtypeStruct((n_pad, 8), jnp.float32)),
  )


def _fuse_tc(n_pad, n, blk):
  nb = n_pad // blk

  def body(s0_ref, s1_ref, x_ref, dv_ref, c0_ref, c1_ref,
           w1_ref, b1_ref, w2_ref, b2_ref, st_ref, ws_ref, bs_ref,
           wc1_ref, wc2_ref, bc_ref, out_ref, acc):
    i = pl.program_id(0)

    @pl.when(i == 0)
    def _():
      acc[...] = jnp.zeros_like(acc)

    dv = dv_ref[...]
    dvc = dv.reshape(blk, 1)
    a1 = dvc * (s0_ref[...] + s1_ref[...] + dvc * x_ref[...])
    h1 = jnp.maximum(
        jnp.dot(a1, w1_ref[...], preferred_element_type=jnp.float32)
        + b1_ref[...], 0.0)
    w = dv * (c0_ref[...] + c1_ref[...] + dv) * (1.0 / n)
    acc[...] += jnp.dot(w.reshape(1, blk), h1,
                        preferred_element_type=jnp.float32)

    @pl.when(i == nb - 1)
    def _():
      pooled = jnp.dot(acc[...], w2_ref[...],
                       preferred_element_type=jnp.float32) + b2_ref[...]
      sf = jnp.maximum(
          jnp.dot(st_ref[...], ws_ref[...],
                  preferred_element_type=jnp.float32) + bs_ref[...], 0.0)
      out_ref[...] = (
          jnp.dot(pooled, wc1_ref[...], preferred_element_type=jnp.float32)
          + jnp.dot(sf, wc2_ref[...], preferred_element_type=jnp.float32)
          + bc_ref[...])

  full = lambda shape: pl.BlockSpec(shape, lambda i: tuple(0 for _ in shape))
  return pl.pallas_call(
      body,
      grid=(nb,),
      in_specs=[
          pl.BlockSpec((blk, 8), lambda i: (i, 0)),
          pl.BlockSpec((blk, 8), lambda i: (i + nb, 0)),
          pl.BlockSpec((blk, 8), lambda i: (i, 0)),
          pl.BlockSpec((blk,), lambda i: (i,)),
          pl.BlockSpec((blk,), lambda i: (i,)),
          pl.BlockSpec((blk,), lambda i: (i + nb,)),
          full((8, 64)), full((1, 64)),
          full((64, 64)), full((1, 64)),
          full((1, 8)), full((8, 64)), full((1, 64)),
          full((64, 2)), full((64, 2)), full((1, 2)),
      ],
      out_specs=pl.BlockSpec((1, 2), lambda i: (0, 0)),
      out_shape=jax.ShapeDtypeStruct((1, 2), jnp.float32),
      scratch_shapes=[pltpu.VMEM((1, 64), jnp.float32)],
  )


def kernel(x, edge_index, state, W1, b1, W2, b2, Ws, bs, Wc, bc):
  n, _ = x.shape
  e = edge_index.shape[1]
  n_pad = ((n + 1 + 1023) // 1024) * 1024
  rows_pt8 = (e + NW * LANES - 1) // (NW * LANES)
  rows_pt = ((rows_pt8 + 7) // 8) * 8
  e_pad = NW * rows_pt * LANES
  rb = 8

  sent = (n + (jnp.arange(e_pad - e, dtype=jnp.int32) % (n_pad - n)))
  src2d = jnp.concatenate([edge_index[0], sent]).reshape(-1, LANES)
  dst2d = jnp.concatenate([edge_index[1], sent]).reshape(-1, LANES)
  x_pad = jnp.pad(x, ((0, n_pad - n), (0, 0)))
  zeros8 = jnp.zeros((_stage_of(n_pad // NS), 8), jnp.float32)

  degp = _deg_kernel(n_pad, rows_pt, 16)(dst2d)

  blk = next(n_pad // nb for nb in (14, 16, 8, 4, 2, 1)
             if n_pad % (nb * 1024) == 0)
  dinv, y = _norm_tc(n_pad, n, blk)(degp, degp, x_pad)

  sp, cp = _main_kernel(n_pad, rows_pt, rb)(src2d, dst2d, y, dinv, zeros8)

  out = _fuse_tc(n_pad, n, blk)(
      sp, sp, x_pad, dinv, cp, cp,
      W1, b1.reshape(1, -1), W2, b2.reshape(1, -1),
      state, Ws, bs.reshape(1, -1), Wc[:64], Wc[64:], bc.reshape(1, -1))
  return out
